# baseline (speedup 1.0000x reference)
import functools
import math

import jax
import jax.numpy as jnp
from jax import lax
from jax.experimental import pallas as pl
from jax.experimental.pallas import tpu as pltpu
from jax.experimental.pallas import tpu_sc as plsc

_FC = 256
_NG = 64
_INV = float((1.0 + 1e-5) ** -0.5)
_LN2 = math.log(2.0)
_NC, _NS = 2, 16
_NW = _NC * _NS


def _silu(v):
    return v * jax.nn.sigmoid(v)


def _dot(a, b):
    return jnp.dot(a, b, preferred_element_type=jnp.float32)


def _edge_embed_body(d_ref, we_ref, be_ref, o_ref):
    d = d_ref[...]
    j = lax.broadcasted_iota(jnp.int32, (1, _FC), 1).astype(jnp.float32)
    centers = -4.0 + (8.0 / (_FC - 1)) * j
    gamma = (_FC - 1) / 8.0
    base = gamma * (d - centers)
    r = jnp.exp(-(base * base))
    h = _dot(r, we_ref[...]) + be_ref[...]
    o_ref[...] = _silu(h)


def _embed_proj_body(x_ref, wa_ref, ba_ref, wca_ref, wcb_ref,
                     xf_ref, pa_ref, pb_ref):
    xf = _dot(x_ref[...], wa_ref[...]) + ba_ref[...]
    xf_ref[...] = xf
    pa_ref[...] = _dot(xf, wca_ref[...]).astype(jnp.bfloat16)
    pb_ref[...] = _dot(xf, wcb_ref[...]).astype(jnp.bfloat16)


def _update_proj_body(xfp_ref, agg_ref, g_ref, b_ref, wca_ref, wcb_ref,
                      xf_ref, pa_ref, pb_ref):
    xf = jnp.maximum(
        xfp_ref[...] + agg_ref[...] * _INV * g_ref[...] + b_ref[...], 0.0)
    xf_ref[...] = xf
    pa_ref[...] = _dot(xf, wca_ref[...]).astype(jnp.bfloat16)
    pb_ref[...] = _dot(xf, wcb_ref[...]).astype(jnp.bfloat16)


def _edge_mlp_body(ga_ref, gb_ref, e_ref, wef_ref, w2f_ref, weh_ref, w2_ref,
                   b1f_ref, b2f_ref, b1_ref, b2_ref, gi_ref, bi_ref, o_ref):
    e = e_ref[...]
    ga = ga_ref[...].astype(jnp.float32)
    gb = gb_ref[...].astype(jnp.float32)
    pre_f = (ga[:, :_FC] + gb[:, :_FC]
             + _dot(e, wef_ref[...]) + b1f_ref[...])
    hf = _dot(_silu(pre_f), w2f_ref[...]) + b2f_ref[...]
    score = jax.nn.sigmoid(hf * _INV * gi_ref[...] + bi_ref[...])
    pre = (ga[:, _FC:] + gb[:, _FC:]
           + _dot(e, weh_ref[...]) + b1_ref[...])
    h = _dot(_silu(pre), w2_ref[...]) + b2_ref[...]
    o_ref[...] = score * h


def _final_body(xfp_ref, agg_ref, g_ref, b_ref, batch_ref, wfc_ref, bfc_ref,
                wout_ref, bout_ref, o_ref, sums_ref, cnts_ref, *, nb):
    pid = pl.program_id(0)

    @pl.when(pid == 0)
    def _():
        sums_ref[...] = jnp.zeros_like(sums_ref)
        cnts_ref[...] = jnp.zeros_like(cnts_ref)

    xf = jnp.maximum(
        xfp_ref[...] + agg_ref[...] * _INV * g_ref[...] + b_ref[...], 0.0)
    bblk = batch_ref[0]
    oh = (lax.broadcasted_iota(jnp.int32, (_NG, nb), 0) == bblk)
    ohf = oh.astype(jnp.float32)
    sums_ref[...] += _dot(ohf, xf)
    cnt = jnp.sum(ohf, axis=1, keepdims=True)
    cnts_ref[...] += jnp.broadcast_to(cnt, cnts_ref.shape)

    @pl.when(pid == pl.num_programs(0) - 1)
    def _():
        pooled = sums_ref[...] / jnp.maximum(cnts_ref[:, 0:1], 1.0)
        z = _dot(pooled, wfc_ref[...]) + bfc_ref[...]
        feats = (jnp.maximum(z, 0.0)
                 + jnp.log1p(jnp.exp(-jnp.abs(z))) - _LN2)
        o_ref[...] = _dot(feats, wout_ref[...]) + bout_ref[...]


def _sc_gather(pa, pb, dst, src):
    ne = dst.shape[0]
    d = pa.shape[1]
    per_w = ne // _NW
    c = 40
    n_ch = per_w // c
    mesh = plsc.VectorSubcoreMesh(core_axis_name="c", subcore_axis_name="s")

    @functools.partial(
        pl.kernel, mesh=mesh,
        out_type=[jax.ShapeDtypeStruct((ne, d), jnp.int32),
                  jax.ShapeDtypeStruct((ne, d), jnp.int32)],
        scratch_types=[pltpu.VMEM((per_w,), jnp.int32),
                       pltpu.VMEM((per_w,), jnp.int32),
                       pltpu.VMEM((c, d), jnp.int32),
                       pltpu.VMEM((c, d), jnp.int32),
                       pltpu.VMEM((c, d), jnp.int32),
                       pltpu.VMEM((c, d), jnp.int32)]
                      + [pltpu.SemaphoreType.DMA] * 8,
    )
    def k(pa_h, pb_h, dst_h, src_h, ga_h, gb_h, ia, ib,
          ra0, ra1, rb0, rb1, sga0, sga1, sgb0, sgb1,
          swa0, swa1, swb0, swb1):
        wid = lax.axis_index("s") * _NC + lax.axis_index("c")
        e0 = wid * per_w
        pltpu.sync_copy(dst_h.at[pl.ds(e0, per_w)], ia)
        pltpu.sync_copy(src_h.at[pl.ds(e0, per_w)], ib)
        ras, rbs = (ra0, ra1), (rb0, rb1)
        sgas, sgbs = (sga0, sga1), (sgb0, sgb1)
        swas, swbs = (swa0, swa1), (swb0, swb1)

        def g_descs(i, b):
            return (pltpu.make_async_copy(
                        pa_h.at[ia.at[pl.ds(i * c, c)]], ras[b], sgas[b]),
                    pltpu.make_async_copy(
                        pb_h.at[ib.at[pl.ds(i * c, c)]], rbs[b], sgbs[b]))

        def w_descs(i, b):
            base = e0 + i * c
            return (pltpu.make_async_copy(
                        ras[b], ga_h.at[pl.ds(base, c)], swas[b]),
                    pltpu.make_async_copy(
                        rbs[b], gb_h.at[pl.ds(base, c)], swbs[b]))

        for dsc in g_descs(0, 0):
            dsc.start()

        def body(j, carry):
            for b in (0, 1):
                i = 2 * j + b

                @pl.when(i >= 1)
                def _(i=i, b=b):
                    for dsc in w_descs(i - 1, 1 - b):
                        dsc.wait()

                @pl.when(i + 1 < n_ch)
                def _(i=i, b=b):
                    for dsc in g_descs(i + 1, 1 - b):
                        dsc.start()

                for dsc in g_descs(i, b):
                    dsc.wait()
                for dsc in w_descs(i, b):
                    dsc.start()
            return carry

        lax.fori_loop(0, n_ch // 2, body, 0)
        i_last = n_ch - 1
        for dsc in w_descs(i_last - 1, 1):
            dsc.wait()
        for dsc in g_descs(i_last, 0):
            dsc.wait()
        for dsc in w_descs(i_last, 0):
            dsc.start()
        for dsc in w_descs(i_last, 0):
            dsc.wait()

    return k(pa, pb, dst, src)


def _sc_scatter(msg, dst, n_nodes):
    ne = dst.shape[0]
    half = _FC // _NC
    c2 = 80
    per_sub = ne // _NS
    n_ch = per_sub // c2
    rc = 80
    n_row_ch = n_nodes // rc
    row_iters = (n_row_ch + _NS - 1) // _NS
    z = jnp.zeros((rc, half), jnp.float32)
    dst3 = dst.reshape(_NS, n_ch, 1, c2)
    mesh = plsc.VectorSubcoreMesh(core_axis_name="c", subcore_axis_name="s")

    @functools.partial(
        pl.kernel, mesh=mesh,
        out_type=jax.ShapeDtypeStruct((n_nodes, _FC), jnp.float32),
        scratch_types=[pltpu.VMEM((n_ch, 1, c2), jnp.int32),
                       pltpu.VMEM((c2, half), jnp.float32),
                       pltpu.VMEM((c2, half), jnp.float32),
                       pltpu.VMEM_SHARED((n_nodes, half), jnp.float32)]
                      + [pltpu.SemaphoreType.DMA] * 4,
    )
    def k(msg_h, dst3_h, z_h, agg_h, idx3, mb0, mb1, acc_sh,
          sl0, sl1, ss0, ss1):
        cid = lax.axis_index("c")
        sid = lax.axis_index("s")
        col0 = cid * half
        mbs, sls, sss = (mb0, mb1), (sl0, sl1), (ss0, ss1)

        def l_desc(i, b):
            base = sid * per_sub + i * c2
            return pltpu.make_async_copy(
                msg_h.at[pl.ds(base, c2), pl.ds(col0, half)], mbs[b], sls[b])

        def s_desc(i, b):
            return pltpu.make_async_copy(
                mbs[b], acc_sh.at[idx3.at[i, 0]], sss[b])

        pltpu.sync_copy(dst3_h.at[sid], idx3)
        l_desc(0, 0).start()

        def zbody(i, carry):
            ch = sid + i * _NS

            @pl.when(ch < n_row_ch)
            def _():
                pltpu.sync_copy(z_h, acc_sh.at[pl.ds(ch * rc, rc)])

            return carry

        lax.fori_loop(0, row_iters, zbody, 0)
        plsc.subcore_barrier()

        def body(j, carry):
            for b in (0, 1):
                i = 2 * j + b
                l_desc(i, b).wait()

                @pl.when(i >= 1)
                def _(i=i, b=b):
                    s_desc(i - 1, 1 - b).wait()

                @pl.when(i + 1 < n_ch)
                def _(i=i, b=b):
                    l_desc(i + 1, 1 - b).start()

                pltpu.async_copy(mbs[b], acc_sh.at[idx3.at[i, 0]], sss[b],
                                 add=True)
            return carry

        lax.fori_loop(0, n_ch // 2, body, 0)
        i_last = n_ch - 1
        l_desc(i_last, 0).wait()
        s_desc(i_last - 1, 1).wait()
        pltpu.async_copy(mbs[0], acc_sh.at[idx3.at[i_last, 0]], sss[0], add=True)
        s_desc(i_last, 0).wait()
        plsc.subcore_barrier()

        def wbody(i, carry):
            ch = sid + i * _NS

            @pl.when(ch < n_row_ch)
            def _():
                r0 = ch * rc
                pltpu.sync_copy(
                    acc_sh.at[pl.ds(r0, rc)],
                    agg_h.at[pl.ds(r0, rc), pl.ds(col0, half)])

            return carry

        lax.fori_loop(0, row_iters, wbody, 0)

    return k(msg, dst3, z)


def _edge_embed(edge_attr, w_edge, b_edge):
    ne = edge_attr.shape[0]
    eb = 2000
    return pl.pallas_call(
        _edge_embed_body,
        grid=(ne // eb,),
        in_specs=[pl.BlockSpec((eb, 1), lambda i: (i, 0)),
                  pl.BlockSpec((_FC, _FC), lambda i: (0, 0)),
                  pl.BlockSpec((1, _FC), lambda i: (0, 0))],
        out_specs=pl.BlockSpec((eb, _FC), lambda i: (i, 0)),
        out_shape=jax.ShapeDtypeStruct((ne, _FC), jnp.float32),
    )(edge_attr.reshape(ne, 1), w_edge, b_edge.reshape(1, _FC))


def _proj_specs(n, nb):
    wspec = pl.BlockSpec((_FC, 2 * _FC), lambda i: (0, 0))
    out_specs = [pl.BlockSpec((nb, _FC), lambda i: (i, 0)),
                 pl.BlockSpec((nb, 2 * _FC), lambda i: (i, 0)),
                 pl.BlockSpec((nb, 2 * _FC), lambda i: (i, 0))]
    out_shape = [jax.ShapeDtypeStruct((n, _FC), jnp.float32),
                 jax.ShapeDtypeStruct((n, 2 * _FC), jnp.bfloat16),
                 jax.ShapeDtypeStruct((n, 2 * _FC), jnp.bfloat16)]
    return wspec, out_specs, out_shape


def _embed_proj(xp, wap, b_atom, wca, wcb):
    n = xp.shape[0]
    nb = 1000
    wspec, out_specs, out_shape = _proj_specs(n, nb)
    return pl.pallas_call(
        _embed_proj_body,
        grid=(n // nb,),
        in_specs=[pl.BlockSpec((nb, 128), lambda i: (i, 0)),
                  pl.BlockSpec((128, _FC), lambda i: (0, 0)),
                  pl.BlockSpec((1, _FC), lambda i: (0, 0)),
                  wspec, wspec],
        out_specs=out_specs,
        out_shape=out_shape,
    )(xp, wap, b_atom.reshape(1, _FC), wca, wcb)


def _update_proj(xf, agg, g, b, wca, wcb):
    n = xf.shape[0]
    nb = 1000
    wspec, out_specs, out_shape = _proj_specs(n, nb)
    nspec = pl.BlockSpec((nb, _FC), lambda i: (i, 0))
    vspec = pl.BlockSpec((1, _FC), lambda i: (0, 0))
    return pl.pallas_call(
        _update_proj_body,
        grid=(n // nb,),
        in_specs=[nspec, nspec, vspec, vspec, wspec, wspec],
        out_specs=out_specs,
        out_shape=out_shape,
    )(xf, agg, g.reshape(1, _FC), b.reshape(1, _FC), wca, wcb)


def _edge_mlp(ga, gb, ef, wef, w2f, weh, w2, b1f, b2f, b1, b2, gi, bi):
    ne = ga.shape[0]
    eb = 1600
    gspec = pl.BlockSpec((eb, 2 * _FC), lambda i: (i, 0))
    espec = pl.BlockSpec((eb, _FC), lambda i: (i, 0))
    wspec = pl.BlockSpec((_FC, _FC), lambda i: (0, 0))
    vspec = pl.BlockSpec((1, _FC), lambda i: (0, 0))
    args = [a.reshape(1, _FC) for a in (b1f, b2f, b1, b2, gi, bi)]
    return pl.pallas_call(
        _edge_mlp_body,
        grid=(ne // eb,),
        in_specs=[gspec, gspec, espec, wspec, wspec, wspec, wspec,
                  vspec, vspec, vspec, vspec, vspec, vspec],
        out_specs=pl.BlockSpec((eb, _FC), lambda i: (i, 0)),
        out_shape=jax.ShapeDtypeStruct((ne, _FC), jnp.float32),
    )(ga, gb, ef, wef, w2f, weh, w2, *args)


def _final(xf, agg, g, b, batch, w_fc, b_fc, wout_p, bout_p):
    n = xf.shape[0]
    nb = 1000
    nspec = pl.BlockSpec((nb, _FC), lambda i: (i, 0))
    vspec = pl.BlockSpec((1, _FC), lambda i: (0, 0))
    return pl.pallas_call(
        functools.partial(_final_body, nb=nb),
        grid=(n // nb,),
        in_specs=[nspec, nspec, vspec, vspec,
                  pl.BlockSpec((1, 1, nb), lambda i: (i, 0, 0)),
                  pl.BlockSpec((_FC, _FC), lambda i: (0, 0)),
                  vspec,
                  pl.BlockSpec((_FC, 128), lambda i: (0, 0)),
                  pl.BlockSpec((1, 128), lambda i: (0, 0))],
        out_specs=pl.BlockSpec((_NG, 128), lambda i: (0, 0)),
        out_shape=jax.ShapeDtypeStruct((_NG, 128), jnp.float32),
        scratch_shapes=[pltpu.VMEM((_NG, _FC), jnp.float32),
                        pltpu.VMEM((_NG, 128), jnp.float32)],
    )(xf, agg, g.reshape(1, _FC), b.reshape(1, _FC),
      batch.reshape(n // nb, 1, nb),
      w_fc, b_fc.reshape(1, _FC), wout_p, bout_p)


def kernel(x, edge_index, edge_attr, batch, W_atom, b_atom, W_edge, b_edge,
           conv_W1f, conv_b1f, conv_W2f, conv_b2f, conv_W1, conv_b1,
           conv_W2, conv_b2, bn_int_g, bn_int_b, bn_g, bn_b,
           W_fc, b_fc, W_out, b_out):
    n, a_in = x.shape
    ne = edge_attr.shape[0]
    layers = conv_W1f.shape[0]
    src = edge_index[0]
    dst = edge_index[1]

    xp = jnp.concatenate([x, jnp.zeros((n, 128 - a_in), x.dtype)], axis=1)
    wap = jnp.concatenate(
        [W_atom, jnp.zeros((128 - a_in, _FC), W_atom.dtype)], axis=0)
    wca = jnp.concatenate([conv_W1f[:, :_FC, :], conv_W1[:, :_FC, :]], axis=2)
    wcb = jnp.concatenate([conv_W1f[:, _FC:2 * _FC, :],
                           conv_W1[:, _FC:2 * _FC, :]], axis=2)
    wef = conv_W1f[:, 2 * _FC:, :]
    weh = conv_W1[:, 2 * _FC:, :]
    wout_p = jnp.concatenate(
        [W_out, jnp.zeros((_FC, 127), W_out.dtype)], axis=1)
    bout_p = jnp.concatenate(
        [b_out, jnp.zeros((127,), b_out.dtype)]).reshape(1, 128)

    ef = _edge_embed(edge_attr, W_edge, b_edge)
    xf, pa, pb = _embed_proj(xp, wap, b_atom, wca[0], wcb[0])
    agg = None
    for l in range(layers):
        if l > 0:
            xf, pa, pb = _update_proj(xf, agg, bn_g[l - 1], bn_b[l - 1],
                                      wca[l], wcb[l])
        pav = lax.bitcast_convert_type(pa.reshape(n, _FC, 2), jnp.int32)
        pbv = lax.bitcast_convert_type(pb.reshape(n, _FC, 2), jnp.int32)
        ga, gb = _sc_gather(pav, pbv, dst, src)
        ga = lax.bitcast_convert_type(ga, jnp.bfloat16).reshape(ne, 2 * _FC)
        gb = lax.bitcast_convert_type(gb, jnp.bfloat16).reshape(ne, 2 * _FC)
        msg = _edge_mlp(ga, gb, ef, wef[l], conv_W2f[l], weh[l], conv_W2[l],
                        conv_b1f[l], conv_b2f[l], conv_b1[l], conv_b2[l],
                        bn_int_g[l], bn_int_b[l])
        agg = _sc_scatter(msg, dst, n)

    out = _final(xf, agg, bn_g[layers - 1], bn_b[layers - 1], batch,
                 W_fc, b_fc, wout_p, bout_p)
    return out[:, 0]


# speedup vs baseline: 5.0786x; 5.0786x over previous
import functools
import math

import jax
import jax.numpy as jnp
from jax import lax
from jax.experimental import pallas as pl
from jax.experimental.pallas import tpu as pltpu
from jax.experimental.pallas import tpu_sc as plsc

_FC = 256
_NG = 64
_INV = float((1.0 + 1e-5) ** -0.5)
_LN2 = math.log(2.0)
_NC, _NS = 2, 16
_NW = _NC * _NS


def _silu(v):
    return v * jax.nn.sigmoid(v)


def _dot(a, b):
    return jnp.dot(a, b, preferred_element_type=jnp.float32)


def _pack2(f_part, h_part):
    fb = lax.bitcast_convert_type(f_part, jnp.int32) + 0x8000
    hb = lax.bitcast_convert_type(h_part, jnp.int32) + 0x8000
    hi = jnp.bitwise_and(fb, jnp.int32(-65536))
    lo = lax.shift_right_logical(hb, 16)
    return jnp.bitwise_or(hi, lo)


def _unpack2(packed):
    hi = jnp.bitwise_and(packed, jnp.int32(-65536))
    lo = lax.shift_left(packed, 16)
    return (lax.bitcast_convert_type(hi, jnp.float32),
            lax.bitcast_convert_type(lo, jnp.float32))


def _edge_embed_body(d_ref, we_ref, be_ref, o_ref):
    d = d_ref[...]
    j = lax.broadcasted_iota(jnp.int32, (1, _FC), 1).astype(jnp.float32)
    centers = -4.0 + (8.0 / (_FC - 1)) * j
    gamma = (_FC - 1) / 8.0
    base = gamma * (d - centers)
    r = jnp.exp(-(base * base))
    h = _dot(r, we_ref[...]) + be_ref[...]
    o_ref[...] = _silu(h)


def _embed_proj_body(x_ref, wa_ref, ba_ref, wca_ref, wcb_ref,
                     xf_ref, pa_ref, pb_ref):
    xf = _dot(x_ref[...], wa_ref[...]) + ba_ref[...]
    xf_ref[...] = xf
    pa = _dot(xf, wca_ref[...])
    pa_ref[...] = _pack2(pa[:, :_FC], pa[:, _FC:])
    pb = _dot(xf, wcb_ref[...])
    pb_ref[...] = _pack2(pb[:, :_FC], pb[:, _FC:])


def _update_proj_body(xfp_ref, agg_ref, g_ref, b_ref, wca_ref, wcb_ref,
                      xf_ref, pa_ref, pb_ref):
    xf = jnp.maximum(
        xfp_ref[...] + agg_ref[...] * _INV * g_ref[...] + b_ref[...], 0.0)
    xf_ref[...] = xf
    pa = _dot(xf, wca_ref[...])
    pa_ref[...] = _pack2(pa[:, :_FC], pa[:, _FC:])
    pb = _dot(xf, wcb_ref[...])
    pb_ref[...] = _pack2(pb[:, :_FC], pb[:, _FC:])


def _edge_mlp_body(ga_ref, gb_ref, e_ref, wef_ref, w2f_ref, weh_ref, w2_ref,
                   b1f_ref, b2f_ref, b1_ref, b2_ref, gi_ref, bi_ref, o_ref):
    e = e_ref[...]
    gaf, gah = _unpack2(ga_ref[...])
    gbf, gbh = _unpack2(gb_ref[...])
    pre_f = gaf + gbf + _dot(e, wef_ref[...]) + b1f_ref[...]
    hf = _dot(_silu(pre_f), w2f_ref[...]) + b2f_ref[...]
    score = jax.nn.sigmoid(hf * _INV * gi_ref[...] + bi_ref[...])
    pre = gah + gbh + _dot(e, weh_ref[...]) + b1_ref[...]
    h = _dot(_silu(pre), w2_ref[...]) + b2_ref[...]
    o_ref[...] = score * h


def _final_body(xfp_ref, agg_ref, g_ref, b_ref, batch_ref, wfc_ref, bfc_ref,
                wout_ref, bout_ref, o_ref, sums_ref, cnts_ref, *, nb):
    pid = pl.program_id(0)

    @pl.when(pid == 0)
    def _():
        sums_ref[...] = jnp.zeros_like(sums_ref)
        cnts_ref[...] = jnp.zeros_like(cnts_ref)

    xf = jnp.maximum(
        xfp_ref[...] + agg_ref[...] * _INV * g_ref[...] + b_ref[...], 0.0)
    bblk = batch_ref[0]
    oh = (lax.broadcasted_iota(jnp.int32, (_NG, nb), 0) == bblk)
    ohf = oh.astype(jnp.float32)
    sums_ref[...] += _dot(ohf, xf)
    cnt = jnp.sum(ohf, axis=1, keepdims=True)
    cnts_ref[...] += jnp.broadcast_to(cnt, cnts_ref.shape)

    @pl.when(pid == pl.num_programs(0) - 1)
    def _():
        pooled = sums_ref[...] / jnp.maximum(cnts_ref[:, 0:1], 1.0)
        z = _dot(pooled, wfc_ref[...]) + bfc_ref[...]
        feats = (jnp.maximum(z, 0.0)
                 + jnp.log1p(jnp.exp(-jnp.abs(z))) - _LN2)
        o_ref[...] = _dot(feats, wout_ref[...]) + bout_ref[...]


def _sc_gather(pa, pb, dst, src):
    ne = dst.shape[0]
    d = pa.shape[1]
    per_w = ne // _NW
    c = 40
    n_ch = per_w // c
    mesh = plsc.VectorSubcoreMesh(core_axis_name="c", subcore_axis_name="s")

    @functools.partial(
        pl.kernel, mesh=mesh,
        out_type=[jax.ShapeDtypeStruct((ne, d), jnp.int32),
                  jax.ShapeDtypeStruct((ne, d), jnp.int32)],
        scratch_types=[pltpu.VMEM((per_w,), jnp.int32),
                       pltpu.VMEM((per_w,), jnp.int32),
                       pltpu.VMEM((c, d), jnp.int32),
                       pltpu.VMEM((c, d), jnp.int32),
                       pltpu.VMEM((c, d), jnp.int32),
                       pltpu.VMEM((c, d), jnp.int32)]
                      + [pltpu.SemaphoreType.DMA] * 8,
    )
    def k(pa_h, pb_h, dst_h, src_h, ga_h, gb_h, ia, ib,
          ra0, ra1, rb0, rb1, sga0, sga1, sgb0, sgb1,
          swa0, swa1, swb0, swb1):
        wid = lax.axis_index("s") * _NC + lax.axis_index("c")
        e0 = wid * per_w
        pltpu.sync_copy(dst_h.at[pl.ds(e0, per_w)], ia)
        pltpu.sync_copy(src_h.at[pl.ds(e0, per_w)], ib)
        ras, rbs = (ra0, ra1), (rb0, rb1)
        sgas, sgbs = (sga0, sga1), (sgb0, sgb1)
        swas, swbs = (swa0, swa1), (swb0, swb1)

        def g_descs(i, b):
            return (pltpu.make_async_copy(
                        pa_h.at[ia.at[pl.ds(i * c, c)]], ras[b], sgas[b]),
                    pltpu.make_async_copy(
                        pb_h.at[ib.at[pl.ds(i * c, c)]], rbs[b], sgbs[b]))

        def w_descs(i, b):
            base = e0 + i * c
            return (pltpu.make_async_copy(
                        ras[b], ga_h.at[pl.ds(base, c)], swas[b]),
                    pltpu.make_async_copy(
                        rbs[b], gb_h.at[pl.ds(base, c)], swbs[b]))

        for dsc in g_descs(0, 0):
            dsc.start()

        def body(j, carry):
            for b in (0, 1):
                i = 2 * j + b

                @pl.when(i >= 1)
                def _(i=i, b=b):
                    for dsc in w_descs(i - 1, 1 - b):
                        dsc.wait()

                @pl.when(i + 1 < n_ch)
                def _(i=i, b=b):
                    for dsc in g_descs(i + 1, 1 - b):
                        dsc.start()

                for dsc in g_descs(i, b):
                    dsc.wait()
                for dsc in w_descs(i, b):
                    dsc.start()
            return carry

        lax.fori_loop(0, n_ch // 2, body, 0)
        i_last = n_ch - 1
        for dsc in w_descs(i_last - 1, 1):
            dsc.wait()
        for dsc in g_descs(i_last, 0):
            dsc.wait()
        for dsc in w_descs(i_last, 0):
            dsc.start()
        for dsc in w_descs(i_last, 0):
            dsc.wait()

    return k(pa, pb, dst, src)


def _sc_scatter(msg, dst, n_nodes):
    ne = dst.shape[0]
    half = _FC // _NC
    c2 = 80
    per_sub = ne // _NS
    n_ch = per_sub // c2
    rc = 80
    n_row_ch = n_nodes // rc
    row_iters = (n_row_ch + _NS - 1) // _NS
    z = jnp.zeros((rc, half), jnp.float32)
    dst3 = dst.reshape(_NS, n_ch, 1, c2)
    mesh = plsc.VectorSubcoreMesh(core_axis_name="c", subcore_axis_name="s")

    @functools.partial(
        pl.kernel, mesh=mesh,
        out_type=jax.ShapeDtypeStruct((n_nodes, _FC), jnp.float32),
        scratch_types=[pltpu.VMEM((n_ch, 1, c2), jnp.int32),
                       pltpu.VMEM((c2, half), jnp.float32),
                       pltpu.VMEM((c2, half), jnp.float32),
                       pltpu.VMEM_SHARED((n_nodes, half), jnp.float32)]
                      + [pltpu.SemaphoreType.DMA] * 4,
    )
    def k(msg_h, dst3_h, z_h, agg_h, idx3, mb0, mb1, acc_sh,
          sl0, sl1, ss0, ss1):
        cid = lax.axis_index("c")
        sid = lax.axis_index("s")
        col0 = cid * half
        mbs, sls, sss = (mb0, mb1), (sl0, sl1), (ss0, ss1)

        def l_desc(i, b):
            base = sid * per_sub + i * c2
            return pltpu.make_async_copy(
                msg_h.at[pl.ds(base, c2), pl.ds(col0, half)], mbs[b], sls[b])

        def s_desc(i, b):
            return pltpu.make_async_copy(
                mbs[b], acc_sh.at[idx3.at[i, 0]], sss[b])

        pltpu.sync_copy(dst3_h.at[sid], idx3)
        l_desc(0, 0).start()

        def zbody(i, carry):
            ch = sid + i * _NS

            @pl.when(ch < n_row_ch)
            def _():
                pltpu.sync_copy(z_h, acc_sh.at[pl.ds(ch * rc, rc)])

            return carry

        lax.fori_loop(0, row_iters, zbody, 0)
        plsc.subcore_barrier()

        def body(j, carry):
            for b in (0, 1):
                i = 2 * j + b
                l_desc(i, b).wait()

                @pl.when(i >= 1)
                def _(i=i, b=b):
                    s_desc(i - 1, 1 - b).wait()

                @pl.when(i + 1 < n_ch)
                def _(i=i, b=b):
                    l_desc(i + 1, 1 - b).start()

                pltpu.async_copy(mbs[b], acc_sh.at[idx3.at[i, 0]], sss[b],
                                 add=True)
            return carry

        lax.fori_loop(0, n_ch // 2, body, 0)
        i_last = n_ch - 1
        l_desc(i_last, 0).wait()
        s_desc(i_last - 1, 1).wait()
        pltpu.async_copy(mbs[0], acc_sh.at[idx3.at[i_last, 0]], sss[0], add=True)
        s_desc(i_last, 0).wait()
        plsc.subcore_barrier()

        def wbody(i, carry):
            ch = sid + i * _NS

            @pl.when(ch < n_row_ch)
            def _():
                r0 = ch * rc
                pltpu.sync_copy(
                    acc_sh.at[pl.ds(r0, rc)],
                    agg_h.at[pl.ds(r0, rc), pl.ds(col0, half)])

            return carry

        lax.fori_loop(0, row_iters, wbody, 0)

    return k(msg, dst3, z)


def _edge_embed(edge_attr, w_edge, b_edge):
    ne = edge_attr.shape[0]
    eb = 2000
    return pl.pallas_call(
        _edge_embed_body,
        grid=(ne // eb,),
        in_specs=[pl.BlockSpec((eb, 1), lambda i: (i, 0)),
                  pl.BlockSpec((_FC, _FC), lambda i: (0, 0)),
                  pl.BlockSpec((1, _FC), lambda i: (0, 0))],
        out_specs=pl.BlockSpec((eb, _FC), lambda i: (i, 0)),
        out_shape=jax.ShapeDtypeStruct((ne, _FC), jnp.float32),
    )(edge_attr.reshape(ne, 1), w_edge, b_edge.reshape(1, _FC))


def _proj_specs(n, nb):
    wspec = pl.BlockSpec((_FC, 2 * _FC), lambda i: (0, 0))
    out_specs = [pl.BlockSpec((nb, _FC), lambda i: (i, 0)),
                 pl.BlockSpec((nb, _FC), lambda i: (i, 0)),
                 pl.BlockSpec((nb, _FC), lambda i: (i, 0))]
    out_shape = [jax.ShapeDtypeStruct((n, _FC), jnp.float32),
                 jax.ShapeDtypeStruct((n, _FC), jnp.int32),
                 jax.ShapeDtypeStruct((n, _FC), jnp.int32)]
    return wspec, out_specs, out_shape


def _embed_proj(xp, wap, b_atom, wca, wcb):
    n = xp.shape[0]
    nb = 1000
    wspec, out_specs, out_shape = _proj_specs(n, nb)
    return pl.pallas_call(
        _embed_proj_body,
        grid=(n // nb,),
        in_specs=[pl.BlockSpec((nb, 128), lambda i: (i, 0)),
                  pl.BlockSpec((128, _FC), lambda i: (0, 0)),
                  pl.BlockSpec((1, _FC), lambda i: (0, 0)),
                  wspec, wspec],
        out_specs=out_specs,
        out_shape=out_shape,
    )(xp, wap, b_atom.reshape(1, _FC), wca, wcb)


def _update_proj(xf, agg, g, b, wca, wcb):
    n = xf.shape[0]
    nb = 1000
    wspec, out_specs, out_shape = _proj_specs(n, nb)
    nspec = pl.BlockSpec((nb, _FC), lambda i: (i, 0))
    vspec = pl.BlockSpec((1, _FC), lambda i: (0, 0))
    return pl.pallas_call(
        _update_proj_body,
        grid=(n // nb,),
        in_specs=[nspec, nspec, vspec, vspec, wspec, wspec],
        out_specs=out_specs,
        out_shape=out_shape,
    )(xf, agg, g.reshape(1, _FC), b.reshape(1, _FC), wca, wcb)


def _edge_mlp(ga, gb, ef, wef, w2f, weh, w2, b1f, b2f, b1, b2, gi, bi):
    ne = ga.shape[0]
    eb = 1600
    gspec = pl.BlockSpec((eb, _FC), lambda i: (i, 0))
    espec = pl.BlockSpec((eb, _FC), lambda i: (i, 0))
    wspec = pl.BlockSpec((_FC, _FC), lambda i: (0, 0))
    vspec = pl.BlockSpec((1, _FC), lambda i: (0, 0))
    args = [a.reshape(1, _FC) for a in (b1f, b2f, b1, b2, gi, bi)]
    return pl.pallas_call(
        _edge_mlp_body,
        grid=(ne // eb,),
        in_specs=[gspec, gspec, espec, wspec, wspec, wspec, wspec,
                  vspec, vspec, vspec, vspec, vspec, vspec],
        out_specs=pl.BlockSpec((eb, _FC), lambda i: (i, 0)),
        out_shape=jax.ShapeDtypeStruct((ne, _FC), jnp.float32),
    )(ga, gb, ef, wef, w2f, weh, w2, *args)


def _final(xf, agg, g, b, batch, w_fc, b_fc, wout_p, bout_p):
    n = xf.shape[0]
    nb = 1000
    nspec = pl.BlockSpec((nb, _FC), lambda i: (i, 0))
    vspec = pl.BlockSpec((1, _FC), lambda i: (0, 0))
    return pl.pallas_call(
        functools.partial(_final_body, nb=nb),
        grid=(n // nb,),
        in_specs=[nspec, nspec, vspec, vspec,
                  pl.BlockSpec((1, 1, nb), lambda i: (i, 0, 0)),
                  pl.BlockSpec((_FC, _FC), lambda i: (0, 0)),
                  vspec,
                  pl.BlockSpec((_FC, 128), lambda i: (0, 0)),
                  pl.BlockSpec((1, 128), lambda i: (0, 0))],
        out_specs=pl.BlockSpec((_NG, 128), lambda i: (0, 0)),
        out_shape=jax.ShapeDtypeStruct((_NG, 128), jnp.float32),
        scratch_shapes=[pltpu.VMEM((_NG, _FC), jnp.float32),
                        pltpu.VMEM((_NG, 128), jnp.float32)],
    )(xf, agg, g.reshape(1, _FC), b.reshape(1, _FC),
      batch.reshape(n // nb, 1, nb),
      w_fc, b_fc.reshape(1, _FC), wout_p, bout_p)


def kernel(x, edge_index, edge_attr, batch, W_atom, b_atom, W_edge, b_edge,
           conv_W1f, conv_b1f, conv_W2f, conv_b2f, conv_W1, conv_b1,
           conv_W2, conv_b2, bn_int_g, bn_int_b, bn_g, bn_b,
           W_fc, b_fc, W_out, b_out):
    n, a_in = x.shape
    ne = edge_attr.shape[0]
    layers = conv_W1f.shape[0]
    src = edge_index[0]
    dst = edge_index[1]

    xp = jnp.concatenate([x, jnp.zeros((n, 128 - a_in), x.dtype)], axis=1)
    wap = jnp.concatenate(
        [W_atom, jnp.zeros((128 - a_in, _FC), W_atom.dtype)], axis=0)
    wca = jnp.concatenate([conv_W1f[:, :_FC, :], conv_W1[:, :_FC, :]], axis=2)
    wcb = jnp.concatenate([conv_W1f[:, _FC:2 * _FC, :],
                           conv_W1[:, _FC:2 * _FC, :]], axis=2)
    wef = conv_W1f[:, 2 * _FC:, :]
    weh = conv_W1[:, 2 * _FC:, :]
    wout_p = jnp.concatenate(
        [W_out, jnp.zeros((_FC, 127), W_out.dtype)], axis=1)
    bout_p = jnp.concatenate(
        [b_out, jnp.zeros((127,), b_out.dtype)]).reshape(1, 128)

    ef = _edge_embed(edge_attr, W_edge, b_edge)
    xf, pa, pb = _embed_proj(xp, wap, b_atom, wca[0], wcb[0])
    agg = None
    for l in range(layers):
        if l > 0:
            xf, pa, pb = _update_proj(xf, agg, bn_g[l - 1], bn_b[l - 1],
                                      wca[l], wcb[l])
        ga, gb = _sc_gather(pa, pb, dst, src)
        msg = _edge_mlp(ga, gb, ef, wef[l], conv_W2f[l], weh[l], conv_W2[l],
                        conv_b1f[l], conv_b2f[l], conv_b1[l], conv_b2[l],
                        bn_int_g[l], bn_int_b[l])
        agg = _sc_scatter(msg, dst, n)

    out = _final(xf, agg, bn_g[layers - 1], bn_b[layers - 1], batch,
                 W_fc, b_fc, wout_p, bout_p)
    return out[:, 0]


# speedup vs baseline: 5.3365x; 1.0508x over previous
import functools
import math

import jax
import jax.numpy as jnp
from jax import lax
from jax.experimental import pallas as pl
from jax.experimental.pallas import tpu as pltpu
from jax.experimental.pallas import tpu_sc as plsc

_FC = 256
_NG = 64
_INV = float((1.0 + 1e-5) ** -0.5)
_LN2 = math.log(2.0)
_NC, _NS = 2, 16
_NW = _NC * _NS


def _silu(v):
    return v * jax.nn.sigmoid(v)


def _dot(a, b):
    return jnp.dot(a, b, preferred_element_type=jnp.float32)


def _pack2(f_part, h_part):
    fb = lax.bitcast_convert_type(f_part, jnp.int32) + 0x8000
    hb = lax.bitcast_convert_type(h_part, jnp.int32) + 0x8000
    hi = jnp.bitwise_and(fb, jnp.int32(-65536))
    lo = lax.shift_right_logical(hb, 16)
    return jnp.bitwise_or(hi, lo)


def _unpack2(packed):
    hi = jnp.bitwise_and(packed, jnp.int32(-65536))
    lo = lax.shift_left(packed, 16)
    return (lax.bitcast_convert_type(hi, jnp.float32),
            lax.bitcast_convert_type(lo, jnp.float32))


def _edge_embed_body(d_ref, we_ref, be_ref, o_ref):
    d = d_ref[...]
    j = lax.broadcasted_iota(jnp.int32, (1, _FC), 1).astype(jnp.float32)
    centers = -4.0 + (8.0 / (_FC - 1)) * j
    gamma = (_FC - 1) / 8.0
    base = gamma * (d - centers)
    r = jnp.exp(-(base * base))
    h = _dot(r, we_ref[...]) + be_ref[...]
    o_ref[...] = _silu(h)


def _embed_proj_body(x_ref, wa_ref, ba_ref, wca_ref, wcb_ref,
                     xf_ref, pa_ref, pb_ref):
    xf = _dot(x_ref[...], wa_ref[...]) + ba_ref[...]
    xf_ref[...] = xf
    pa = _dot(xf, wca_ref[...])
    pa_ref[...] = _pack2(pa[:, :_FC], pa[:, _FC:])
    pb = _dot(xf, wcb_ref[...])
    pb_ref[...] = _pack2(pb[:, :_FC], pb[:, _FC:])


def _update_proj_body(xfp_ref, agg1_ref, agg2_ref, g_ref, b_ref,
                      wca_ref, wcb_ref, xf_ref, pa_ref, pb_ref):
    agg = agg1_ref[...] + agg2_ref[...]
    xf = jnp.maximum(
        xfp_ref[...] + agg * _INV * g_ref[...] + b_ref[...], 0.0)
    xf_ref[...] = xf
    pa = _dot(xf, wca_ref[...])
    pa_ref[...] = _pack2(pa[:, :_FC], pa[:, _FC:])
    pb = _dot(xf, wcb_ref[...])
    pb_ref[...] = _pack2(pb[:, :_FC], pb[:, _FC:])


def _edge_mlp_body(ga_ref, gb_ref, e_ref, wef_ref, w2f_ref, weh_ref, w2_ref,
                   b1f_ref, b2f_ref, b1_ref, b2_ref, gi_ref, bi_ref, o_ref):
    e = e_ref[...]
    gaf, gah = _unpack2(ga_ref[...])
    gbf, gbh = _unpack2(gb_ref[...])
    pre_f = gaf + gbf + _dot(e, wef_ref[...]) + b1f_ref[...]
    hf = _dot(_silu(pre_f), w2f_ref[...]) + b2f_ref[...]
    score = jax.nn.sigmoid(hf * _INV * gi_ref[...] + bi_ref[...])
    pre = gah + gbh + _dot(e, weh_ref[...]) + b1_ref[...]
    h = _dot(_silu(pre), w2_ref[...]) + b2_ref[...]
    o_ref[...] = score * h


def _final_body(xfp_ref, agg1_ref, agg2_ref, g_ref, b_ref, batch_ref,
                wfc_ref, bfc_ref, wout_ref, bout_ref, o_ref,
                sums_ref, cnts_ref, *, nb):
    pid = pl.program_id(0)

    @pl.when(pid == 0)
    def _():
        sums_ref[...] = jnp.zeros_like(sums_ref)
        cnts_ref[...] = jnp.zeros_like(cnts_ref)

    agg = agg1_ref[...] + agg2_ref[...]
    xf = jnp.maximum(
        xfp_ref[...] + agg * _INV * g_ref[...] + b_ref[...], 0.0)
    bblk = batch_ref[0]
    oh = (lax.broadcasted_iota(jnp.int32, (_NG, nb), 0) == bblk)
    ohf = oh.astype(jnp.float32)
    sums_ref[...] += _dot(ohf, xf)
    cnt = jnp.sum(ohf, axis=1, keepdims=True)
    cnts_ref[...] += jnp.broadcast_to(cnt, cnts_ref.shape)

    @pl.when(pid == pl.num_programs(0) - 1)
    def _():
        pooled = sums_ref[...] / jnp.maximum(cnts_ref[:, 0:1], 1.0)
        z = _dot(pooled, wfc_ref[...]) + bfc_ref[...]
        feats = (jnp.maximum(z, 0.0)
                 + jnp.log1p(jnp.exp(-jnp.abs(z))) - _LN2)
        o_ref[...] = _dot(feats, wout_ref[...]) + bout_ref[...]


def _sc_gather(pa, pb, dst, src):
    ne = dst.shape[0]
    d = pa.shape[1]
    per_w = ne // _NW
    c = 40
    n_ch = per_w // c
    mesh = plsc.VectorSubcoreMesh(core_axis_name="c", subcore_axis_name="s")

    @functools.partial(
        pl.kernel, mesh=mesh,
        out_type=[jax.ShapeDtypeStruct((ne, d), jnp.int32),
                  jax.ShapeDtypeStruct((ne, d), jnp.int32)],
        scratch_types=[pltpu.VMEM((per_w,), jnp.int32),
                       pltpu.VMEM((per_w,), jnp.int32),
                       pltpu.VMEM((c, d), jnp.int32),
                       pltpu.VMEM((c, d), jnp.int32),
                       pltpu.VMEM((c, d), jnp.int32),
                       pltpu.VMEM((c, d), jnp.int32)]
                      + [pltpu.SemaphoreType.DMA] * 8,
    )
    def k(pa_h, pb_h, dst_h, src_h, ga_h, gb_h, ia, ib,
          ra0, ra1, rb0, rb1, sga0, sga1, sgb0, sgb1,
          swa0, swa1, swb0, swb1):
        wid = lax.axis_index("s") * _NC + lax.axis_index("c")
        e0 = wid * per_w
        pltpu.sync_copy(dst_h.at[pl.ds(e0, per_w)], ia)
        pltpu.sync_copy(src_h.at[pl.ds(e0, per_w)], ib)
        ras, rbs = (ra0, ra1), (rb0, rb1)
        sgas, sgbs = (sga0, sga1), (sgb0, sgb1)
        swas, swbs = (swa0, swa1), (swb0, swb1)

        def g_descs(i, b):
            return (pltpu.make_async_copy(
                        pa_h.at[ia.at[pl.ds(i * c, c)]], ras[b], sgas[b]),
                    pltpu.make_async_copy(
                        pb_h.at[ib.at[pl.ds(i * c, c)]], rbs[b], sgbs[b]))

        def w_descs(i, b):
            base = e0 + i * c
            return (pltpu.make_async_copy(
                        ras[b], ga_h.at[pl.ds(base, c)], swas[b]),
                    pltpu.make_async_copy(
                        rbs[b], gb_h.at[pl.ds(base, c)], swbs[b]))

        for dsc in g_descs(0, 0):
            dsc.start()

        def body(j, carry):
            for b in (0, 1):
                i = 2 * j + b

                @pl.when(i >= 1)
                def _(i=i, b=b):
                    for dsc in w_descs(i - 1, 1 - b):
                        dsc.wait()

                @pl.when(i + 1 < n_ch)
                def _(i=i, b=b):
                    for dsc in g_descs(i + 1, 1 - b):
                        dsc.start()

                for dsc in g_descs(i, b):
                    dsc.wait()
                for dsc in w_descs(i, b):
                    dsc.start()
            return carry

        lax.fori_loop(0, n_ch // 2, body, 0)
        if n_ch % 2:
            i_last = n_ch - 1
            for dsc in w_descs(i_last - 1, 1):
                dsc.wait()
            for dsc in g_descs(i_last, 0):
                dsc.wait()
            for dsc in w_descs(i_last, 0):
                dsc.start()
            for dsc in w_descs(i_last, 0):
                dsc.wait()
        else:
            for dsc in w_descs(n_ch - 1, 1):
                dsc.wait()

    return k(pa, pb, dst, src)


def _sc_scatter(msg, dst, n_nodes):
    ne = dst.shape[0]
    half = _FC // _NC
    c2 = 80
    per_sub = ne // _NS
    n_ch = per_sub // c2
    rc = 80
    n_row_ch = n_nodes // rc
    row_iters = (n_row_ch + _NS - 1) // _NS
    z = jnp.zeros((rc, half), jnp.float32)
    dst3 = dst.reshape(_NS, n_ch, 1, c2)
    mesh = plsc.VectorSubcoreMesh(core_axis_name="c", subcore_axis_name="s")

    @functools.partial(
        pl.kernel, mesh=mesh,
        out_type=jax.ShapeDtypeStruct((n_nodes, _FC), jnp.float32),
        scratch_types=[pltpu.VMEM((n_ch, 1, c2), jnp.int32),
                       pltpu.VMEM((c2, half), jnp.float32),
                       pltpu.VMEM((c2, half), jnp.float32),
                       pltpu.VMEM_SHARED((n_nodes, half), jnp.float32)]
                      + [pltpu.SemaphoreType.DMA] * 4,
    )
    def k(msg_h, dst3_h, z_h, agg_h, idx3, mb0, mb1, acc_sh,
          sl0, sl1, ss0, ss1):
        cid = lax.axis_index("c")
        sid = lax.axis_index("s")
        col0 = cid * half
        mbs, sls, sss = (mb0, mb1), (sl0, sl1), (ss0, ss1)

        def l_desc(i, b):
            base = sid * per_sub + i * c2
            return pltpu.make_async_copy(
                msg_h.at[pl.ds(base, c2), pl.ds(col0, half)], mbs[b], sls[b])

        def s_desc(i, b):
            return pltpu.make_async_copy(
                mbs[b], acc_sh.at[idx3.at[i, 0]], sss[b])

        pltpu.sync_copy(dst3_h.at[sid], idx3)
        l_desc(0, 0).start()

        def zbody(i, carry):
            ch = sid + i * _NS

            @pl.when(ch < n_row_ch)
            def _():
                pltpu.sync_copy(z_h, acc_sh.at[pl.ds(ch * rc, rc)])

            return carry

        lax.fori_loop(0, row_iters, zbody, 0)
        plsc.subcore_barrier()

        def body(j, carry):
            for b in (0, 1):
                i = 2 * j + b
                l_desc(i, b).wait()

                @pl.when(i >= 1)
                def _(i=i, b=b):
                    s_desc(i - 1, 1 - b).wait()

                @pl.when(i + 1 < n_ch)
                def _(i=i, b=b):
                    l_desc(i + 1, 1 - b).start()

                pltpu.async_copy(mbs[b], acc_sh.at[idx3.at[i, 0]], sss[b],
                                 add=True)
            return carry

        lax.fori_loop(0, n_ch // 2, body, 0)
        if n_ch % 2:
            i_last = n_ch - 1
            l_desc(i_last, 0).wait()
            s_desc(i_last - 1, 1).wait()
            pltpu.async_copy(mbs[0], acc_sh.at[idx3.at[i_last, 0]], sss[0],
                             add=True)
            s_desc(i_last, 0).wait()
        else:
            s_desc(n_ch - 1, 1).wait()
        plsc.subcore_barrier()

        def wbody(i, carry):
            ch = sid + i * _NS

            @pl.when(ch < n_row_ch)
            def _():
                r0 = ch * rc
                pltpu.sync_copy(
                    acc_sh.at[pl.ds(r0, rc)],
                    agg_h.at[pl.ds(r0, rc), pl.ds(col0, half)])

            return carry

        lax.fori_loop(0, row_iters, wbody, 0)

    return k(msg, dst3, z)


def _edge_embed(edge_attr, w_edge, b_edge):
    ne = edge_attr.shape[0]
    eb = 2000
    return pl.pallas_call(
        _edge_embed_body,
        grid=(ne // eb,),
        in_specs=[pl.BlockSpec((eb, 1), lambda i: (i, 0)),
                  pl.BlockSpec((_FC, _FC), lambda i: (0, 0)),
                  pl.BlockSpec((1, _FC), lambda i: (0, 0))],
        out_specs=pl.BlockSpec((eb, _FC), lambda i: (i, 0)),
        out_shape=jax.ShapeDtypeStruct((ne, _FC), jnp.float32),
    )(edge_attr.reshape(ne, 1), w_edge, b_edge.reshape(1, _FC))


def _proj_specs(n, nb):
    wspec = pl.BlockSpec((_FC, 2 * _FC), lambda i: (0, 0))
    out_specs = [pl.BlockSpec((nb, _FC), lambda i: (i, 0)),
                 pl.BlockSpec((nb, _FC), lambda i: (i, 0)),
                 pl.BlockSpec((nb, _FC), lambda i: (i, 0))]
    out_shape = [jax.ShapeDtypeStruct((n, _FC), jnp.float32),
                 jax.ShapeDtypeStruct((n, _FC), jnp.int32),
                 jax.ShapeDtypeStruct((n, _FC), jnp.int32)]
    return wspec, out_specs, out_shape


def _embed_proj(xp, wap, b_atom, wca, wcb):
    n = xp.shape[0]
    nb = 1000
    wspec, out_specs, out_shape = _proj_specs(n, nb)
    return pl.pallas_call(
        _embed_proj_body,
        grid=(n // nb,),
        in_specs=[pl.BlockSpec((nb, 128), lambda i: (i, 0)),
                  pl.BlockSpec((128, _FC), lambda i: (0, 0)),
                  pl.BlockSpec((1, _FC), lambda i: (0, 0)),
                  wspec, wspec],
        out_specs=out_specs,
        out_shape=out_shape,
    )(xp, wap, b_atom.reshape(1, _FC), wca, wcb)


def _update_proj(xf, agg1, agg2, g, b, wca, wcb):
    n = xf.shape[0]
    nb = 1000
    wspec, out_specs, out_shape = _proj_specs(n, nb)
    nspec = pl.BlockSpec((nb, _FC), lambda i: (i, 0))
    vspec = pl.BlockSpec((1, _FC), lambda i: (0, 0))
    return pl.pallas_call(
        _update_proj_body,
        grid=(n // nb,),
        in_specs=[nspec, nspec, nspec, vspec, vspec, wspec, wspec],
        out_specs=out_specs,
        out_shape=out_shape,
    )(xf, agg1, agg2, g.reshape(1, _FC), b.reshape(1, _FC), wca, wcb)


def _edge_mlp(ga, gb, ef, off, wef, w2f, weh, w2, b1f, b2f, b1, b2, gi, bi):
    ne = ga.shape[0]
    eb = 1600
    gspec = pl.BlockSpec((eb, _FC), lambda i: (i, 0))
    espec = pl.BlockSpec((eb, _FC), lambda i, off=off: (i + off, 0))
    wspec = pl.BlockSpec((_FC, _FC), lambda i: (0, 0))
    vspec = pl.BlockSpec((1, _FC), lambda i: (0, 0))
    args = [a.reshape(1, _FC) for a in (b1f, b2f, b1, b2, gi, bi)]
    return pl.pallas_call(
        _edge_mlp_body,
        grid=(ne // eb,),
        in_specs=[gspec, gspec, espec, wspec, wspec, wspec, wspec,
                  vspec, vspec, vspec, vspec, vspec, vspec],
        out_specs=pl.BlockSpec((eb, _FC), lambda i: (i, 0)),
        out_shape=jax.ShapeDtypeStruct((ne, _FC), jnp.float32),
    )(ga, gb, ef, wef, w2f, weh, w2, *args)


def _final(xf, agg1, agg2, g, b, batch, w_fc, b_fc, wout_p, bout_p):
    n = xf.shape[0]
    nb = 1000
    nspec = pl.BlockSpec((nb, _FC), lambda i: (i, 0))
    vspec = pl.BlockSpec((1, _FC), lambda i: (0, 0))
    return pl.pallas_call(
        functools.partial(_final_body, nb=nb),
        grid=(n // nb,),
        in_specs=[nspec, nspec, nspec, vspec, vspec,
                  pl.BlockSpec((1, 1, nb), lambda i: (i, 0, 0)),
                  pl.BlockSpec((_FC, _FC), lambda i: (0, 0)),
                  vspec,
                  pl.BlockSpec((_FC, 128), lambda i: (0, 0)),
                  pl.BlockSpec((1, 128), lambda i: (0, 0))],
        out_specs=pl.BlockSpec((_NG, 128), lambda i: (0, 0)),
        out_shape=jax.ShapeDtypeStruct((_NG, 128), jnp.float32),
        scratch_shapes=[pltpu.VMEM((_NG, _FC), jnp.float32),
                        pltpu.VMEM((_NG, 128), jnp.float32)],
    )(xf, agg1, agg2, g.reshape(1, _FC), b.reshape(1, _FC),
      batch.reshape(n // nb, 1, nb),
      w_fc, b_fc.reshape(1, _FC), wout_p, bout_p)


def kernel(x, edge_index, edge_attr, batch, W_atom, b_atom, W_edge, b_edge,
           conv_W1f, conv_b1f, conv_W2f, conv_b2f, conv_W1, conv_b1,
           conv_W2, conv_b2, bn_int_g, bn_int_b, bn_g, bn_b,
           W_fc, b_fc, W_out, b_out):
    n, a_in = x.shape
    ne = edge_attr.shape[0]
    layers = conv_W1f.shape[0]
    src = edge_index[0]
    dst = edge_index[1]

    xp = jnp.concatenate([x, jnp.zeros((n, 128 - a_in), x.dtype)], axis=1)
    wap = jnp.concatenate(
        [W_atom, jnp.zeros((128 - a_in, _FC), W_atom.dtype)], axis=0)
    wca = jnp.concatenate([conv_W1f[:, :_FC, :], conv_W1[:, :_FC, :]], axis=2)
    wcb = jnp.concatenate([conv_W1f[:, _FC:2 * _FC, :],
                           conv_W1[:, _FC:2 * _FC, :]], axis=2)
    wef = conv_W1f[:, 2 * _FC:, :]
    weh = conv_W1[:, 2 * _FC:, :]
    wout_p = jnp.concatenate(
        [W_out, jnp.zeros((_FC, 127), W_out.dtype)], axis=1)
    bout_p = jnp.concatenate(
        [b_out, jnp.zeros((127,), b_out.dtype)]).reshape(1, 128)

    s1 = (ne * 3 // 5) // 1280 * 1280
    halves = ((0, s1), (s1, ne))
    dsts = [dst[a:b] for a, b in halves]
    srcs = [src[a:b] for a, b in halves]

    ef = _edge_embed(edge_attr, W_edge, b_edge)
    xf, pa, pb = _embed_proj(xp, wap, b_atom, wca[0], wcb[0])
    aggs = None
    for l in range(layers):
        if l > 0:
            xf, pa, pb = _update_proj(xf, aggs[0], aggs[1],
                                      bn_g[l - 1], bn_b[l - 1],
                                      wca[l], wcb[l])
        gs = [_sc_gather(pa, pb, dsts[h], srcs[h]) for h in (0, 1)]
        msgs = [_edge_mlp(gs[h][0], gs[h][1], ef, halves[h][0] // 1600,
                          wef[l], conv_W2f[l], weh[l], conv_W2[l],
                          conv_b1f[l], conv_b2f[l], conv_b1[l], conv_b2[l],
                          bn_int_g[l], bn_int_b[l]) for h in (0, 1)]
        aggs = [_sc_scatter(msgs[h], dsts[h], n) for h in (0, 1)]

    out = _final(xf, aggs[0], aggs[1], bn_g[layers - 1], bn_b[layers - 1],
                 batch, W_fc, b_fc, wout_p, bout_p)
    return out[:, 0]
